# baseline (speedup 1.0000x reference)
import functools

import jax
import jax.numpy as jnp
from jax import lax
from jax.experimental import pallas as pl
from jax.experimental.pallas import tpu as pltpu
from jax.experimental.pallas import tpu_sc as plsc

EMBED_DIM = 32
HALF = 16
LANES = 16

_NC = 2
_NS = 16
_NW = _NC * _NS

_T = 200
_I = 16384
_JB = 8
_CI = 4096
_NCH = _I // _CI
_UNITS = _T * (EMBED_DIM // _JB)
_UPW = _UNITS // _NW
_STEPS = _UPW * _NCH
_GROUPS = _CI // LANES


def _vperm(src, idx):
    return lax.gather(
        src, idx[:, None],
        dimension_numbers=lax.GatherDimensionNumbers(
            offset_dims=(), collapsed_slice_dims=(0,), start_index_map=(0,)),
        slice_sizes=(1,),
        mode=lax.GatherScatterMode.PROMISE_IN_BOUNDS)


def _sc_lookup(daysT_flat, dtT_flat, w_vec, b_vec):
    mesh = plsc.VectorSubcoreMesh(
        core_axis_name="c", subcore_axis_name="s",
        num_cores=_NC, num_subcores=_NS)

    @functools.partial(
        pl.kernel,
        out_type=jax.ShapeDtypeStruct((_T, EMBED_DIM, _I), jnp.float32),
        mesh=mesh,
        scratch_types=[
            pltpu.VMEM((_CI,), jnp.int32),
            pltpu.VMEM((_CI,), jnp.int32),
            pltpu.VMEM((_JB, _CI), jnp.float32),
            pltpu.VMEM((_JB, _CI), jnp.float32),
            pltpu.VMEM((EMBED_DIM * LANES,), jnp.float32),
            pltpu.VMEM((LANES,), jnp.float32),
            pltpu.VMEM((LANES,), jnp.float32),
            pltpu.SemaphoreType.DMA,
            pltpu.SemaphoreType.DMA,
            pltpu.SemaphoreType.DMA,
            pltpu.SemaphoreType.DMA,
        ],
        compiler_params=pltpu.CompilerParams(needs_layout_passes=False),
    )
    def k(idx_hbm, tbl_hbm, w_hbm, b_hbm, out_hbm,
          idx_v0, idx_v1, out_v0, out_v1, fused_v, w_v, b_v,
          sem_in0, sem_in1, sem_out0, sem_out1):
        idx_v = (idx_v0, idx_v1)
        out_v = (out_v0, out_v1)
        sem_in = (sem_in0, sem_in1)
        sem_out = (sem_out0, sem_out1)
        wid = lax.axis_index("s") * _NC + lax.axis_index("c")
        u_base = wid * _UPW

        pltpu.sync_copy(tbl_hbm, fused_v.at[pl.ds(0, HALF * LANES)])
        pltpu.sync_copy(w_hbm, w_v)
        pltpu.sync_copy(b_hbm, b_v)
        w_all = w_v[...]
        b_all = b_v[...]
        wk = jnp.where(lax.iota(jnp.int32, LANES) >= 5, 1.0, 0.0)
        for j in range(HALF):
            sel = jnp.full((LANES,), j, jnp.int32)
            fused_v[pl.ds((HALF + j) * LANES, LANES)] = (
                wk * _vperm(w_all, sel) + _vperm(b_all, sel))

        def unit_of(s):
            u = u_base + (s >> 2)
            return u >> 2, u & 3, s & 3

        def in_cp(s, b):
            t, _, ch = unit_of(s)
            off = pl.multiple_of(t * _I + ch * _CI, _CI)
            return pltpu.make_async_copy(
                idx_hbm.at[pl.ds(off, _CI)], idx_v[b], sem_in[b])

        def out_cp(s, b):
            t, jb, ch = unit_of(s)
            return pltpu.make_async_copy(
                out_v[b],
                out_hbm.at[t, pl.ds(jb * _JB, _JB),
                           pl.ds(pl.multiple_of(ch * _CI, _CI), _CI)],
                sem_out[b])

        in_cp(0, 0).start()
        in_cp(1, 1).start()

        def step_pair(K, carry):
            for b in range(2):
                s = 2 * K + b
                in_cp(s, b).wait()

                @pl.when(K >= 1)
                def _():
                    out_cp(s - 2, b).wait()

                _, jb, _ = unit_of(s)
                cbase = pl.multiple_of(jb * (_JB * LANES), _JB * LANES)
                colv = [fused_v[pl.ds(cbase + jj * LANES, LANES)]
                        for jj in range(_JB)]
                idx_ref = idx_v[b]
                out_ref = out_v[b]

                def group(g, carry2):
                    span = pl.ds(pl.multiple_of(g * LANES, LANES), LANES)
                    dvec = idx_ref[span]
                    for jj in range(_JB):
                        out_ref[jj, span] = _vperm(colv[jj], dvec)
                    return carry2

                lax.fori_loop(0, _GROUPS, group, 0)
                out_cp(s, b).start()

                @pl.when(K < _STEPS // 2 - 1)
                def _():
                    in_cp(s + 2, b).start()
            return carry

        lax.fori_loop(0, _STEPS // 2, step_pair, 0)
        out_cp(_STEPS - 2, 0).wait()
        out_cp(_STEPS - 1, 1).wait()

    return k(daysT_flat, dtT_flat, w_vec, b_vec)


def kernel(days, day_table, W_weekend, b_weekend):
    daysT_flat = days.T.reshape(_T * _I)
    dtT_flat = jnp.zeros((LANES, LANES), jnp.float32).at[:, :7].set(
        day_table.T).reshape(HALF * LANES)
    a = _sc_lookup(daysT_flat, dtT_flat, W_weekend.reshape(HALF),
                   b_weekend.reshape(HALF))
    return a.transpose(2, 0, 1)
